# baseline (speedup 1.0000x reference)
import functools

import jax
import jax.numpy as jnp
from jax import lax
from jax.experimental import pallas as pl
from jax.experimental.pallas import tpu as pltpu
from jax.experimental.pallas import tpu_sc as plsc

NUM_CORES = 2
NUM_SUBCORES = 16
NW = NUM_CORES * NUM_SUBCORES
LANES = 16

SEQ_L = 200
BATCH = 4096
N = SEQ_L * BATCH
D = 64
N_W = N // NW
CHUNK = 128
G = N_W // CHUNK
NBUF = 4
SCALE = 8.0

_mesh = plsc.VectorSubcoreMesh(core_axis_name="c", subcore_axis_name="s")


@functools.partial(
    pl.kernel,
    out_type=jax.ShapeDtypeStruct((SEQ_L, D // 8, BATCH // CHUNK, 8, CHUNK),
                                  jnp.float32),
    mesh=_mesh,
    scratch_types=[
        pltpu.VMEM((G, CHUNK), jnp.int32),
        [pltpu.VMEM((CHUNK, D), jnp.float32) for _ in range(NBUF)],
        [pltpu.VMEM((D, CHUNK), jnp.float32) for _ in range(NBUF)],
        [pltpu.SemaphoreType.DMA for _ in range(NBUF)],
        [pltpu.SemaphoreType.DMA for _ in range(NBUF)],
    ],
    compiler_params=pltpu.CompilerParams(
        use_tc_tiling_on_sc=False, needs_layout_passes=False
    ),
)
def _emb_lookup(idx_hbm, table_hbm, out_hbm, idx_v, bufs, obufs, gsems, ssems):
    wid = lax.axis_index("s") * NUM_CORES + lax.axis_index("c")
    base = wid * N_W

    pltpu.sync_copy(idx_hbm.at[wid], idx_v)

    rows16 = [jnp.arange(16, dtype=jnp.int32) + 16 * rg for rg in range(8)]

    def start(g, b):
        pltpu.async_copy(table_hbm.at[idx_v.at[g]], bufs[b], gsems[b])

    def out_slab(g, te):
        p0 = base + g * CHUNK
        l = p0 // BATCH
        tb = (p0 % BATCH) // CHUNK
        return out_hbm.at[l, te, tb]

    def drain_stores(g, b):
        for te in range(8):
            pltpu.make_async_copy(
                obufs[b].at[pl.ds(te * 8, 8)], out_slab(g, te), ssems[b]
            ).wait()

    def finish(g, b):
        pltpu.make_async_copy(table_hbm.at[idx_v.at[g]], bufs[b], gsems[b]).wait()

        def col(e, _):
            for rg in range(8):
                vals = plsc.load_gather(
                    bufs[b], [rows16[rg], jnp.full((16,), e, jnp.int32)]
                )
                obufs[b][e, pl.ds(16 * rg, 16)] = vals * SCALE
            return 0

        lax.fori_loop(0, D, col, 0)

        for te in range(8):
            pltpu.async_copy(
                obufs[b].at[pl.ds(te * 8, 8)], out_slab(g, te), ssems[b]
            )

    for b in range(NBUF):
        start(b, b)

    def group(k, _):
        for b in range(NBUF):
            g = k * NBUF + b
            pl.when(k > 0)(lambda: drain_stores(g - NBUF, b))
            finish(g, b)
            start(g + NBUF, b)
        return 0

    lax.fori_loop(0, G // NBUF - 1, group, 0)

    k_last = G // NBUF - 1
    for b in range(NBUF):
        g = k_last * NBUF + b
        drain_stores(g - NBUF, b)
        finish(g, b)
        drain_stores(g, b)


def kernel(input_, table):
    idx = input_.reshape(NW, G, CHUNK)
    out5 = _emb_lookup(idx, table)
    return out5.transpose(0, 2, 4, 1, 3).reshape(SEQ_L, BATCH, D)


# speedup vs baseline: 2.5839x; 2.5839x over previous
import functools

import jax
import jax.numpy as jnp
from jax import lax
from jax.experimental import pallas as pl
from jax.experimental.pallas import tpu as pltpu
from jax.experimental.pallas import tpu_sc as plsc

NUM_CORES = 2
NUM_SUBCORES = 16
NW = NUM_CORES * NUM_SUBCORES
LANES = 16

SEQ_L = 200
BATCH = 4096
N = SEQ_L * BATCH
D = 64
N_W = N // NW
CHUNK = 128
G = N_W // CHUNK
NBUF = 4
SCALE = 8.0

_mesh = plsc.VectorSubcoreMesh(core_axis_name="c", subcore_axis_name="s")


@functools.partial(
    pl.kernel,
    out_type=jax.ShapeDtypeStruct((SEQ_L, D // 8, BATCH // CHUNK, 8, CHUNK),
                                  jnp.float32),
    mesh=_mesh,
    scratch_types=[
        pltpu.VMEM((G, CHUNK), jnp.int32),
        [pltpu.VMEM((CHUNK, D), jnp.float32) for _ in range(NBUF)],
        [pltpu.VMEM((D, CHUNK + 1), jnp.float32) for _ in range(NBUF)],
        [pltpu.SemaphoreType.DMA for _ in range(NBUF)],
        [pltpu.SemaphoreType.DMA for _ in range(NBUF)],
    ],
    compiler_params=pltpu.CompilerParams(
        use_tc_tiling_on_sc=False, needs_layout_passes=False
    ),
)
def _emb_lookup(idx_hbm, table_hbm, out_hbm, idx_v, bufs, obufs, gsems, ssems):
    wid = lax.axis_index("s") * NUM_CORES + lax.axis_index("c")
    base = wid * N_W

    pltpu.sync_copy(idx_hbm.at[wid], idx_v)

    cols16 = [jnp.arange(16, dtype=jnp.int32) + 16 * c for c in range(D // LANES)]

    def start(g, b):
        pltpu.async_copy(table_hbm.at[idx_v.at[g]], bufs[b], gsems[b])

    def out_slab(g, te):
        p0 = base + g * CHUNK
        l = p0 // BATCH
        tb = (p0 % BATCH) // CHUNK
        return out_hbm.at[l, te, tb]

    def drain_stores(g, b):
        for te in range(8):
            pltpu.make_async_copy(
                obufs[b].at[pl.ds(te * 8, 8), pl.ds(0, CHUNK)],
                out_slab(g, te),
                ssems[b],
            ).wait()

    def finish(g, b):
        pltpu.make_async_copy(table_hbm.at[idx_v.at[g]], bufs[b], gsems[b]).wait()

        @plsc.parallel_loop(0, CHUNK, unroll=4)
        def _(r):
            rvec = jnp.full((LANES,), r, jnp.int32)
            for c in range(D // LANES):
                vals = bufs[b][r, pl.ds(16 * c, LANES)]
                plsc.store_scatter(obufs[b], [cols16[c], rvec], vals * SCALE)

        for te in range(8):
            pltpu.async_copy(
                obufs[b].at[pl.ds(te * 8, 8), pl.ds(0, CHUNK)],
                out_slab(g, te),
                ssems[b],
            )

    for b in range(NBUF):
        start(b, b)

    def group(k, _):
        for b in range(NBUF):
            g = k * NBUF + b
            pl.when(k > 0)(lambda: drain_stores(g - NBUF, b))
            finish(g, b)
            start(g + NBUF, b)
        return 0

    lax.fori_loop(0, G // NBUF - 1, group, 0)

    k_last = G // NBUF - 1
    for b in range(NBUF):
        g = k_last * NBUF + b
        drain_stores(g - NBUF, b)
        finish(g, b)
        drain_stores(g, b)


def kernel(input_, table):
    idx = input_.reshape(NW, G, CHUNK)
    out5 = _emb_lookup(idx, table)
    return out5.transpose(0, 2, 4, 1, 3).reshape(SEQ_L, BATCH, D)
